# baseline (speedup 1.0000x reference)
import functools

import jax
import jax.numpy as jnp
from jax import lax
from jax.experimental import pallas as pl
from jax.experimental.pallas import tpu as pltpu
from jax.experimental.pallas import tpu_sc as plsc

_NC, _NS = 2, 16
_NW = _NC * _NS
_CHUNK = 128
_IBR = 8
_RCH = 128


def _prep_idx(src, dst, nin, nout):
    n = 27 * nout
    nblk = -(-n // (_NW * _CHUNK))
    nblk = -(-nblk // _IBR) * _IBR
    n_pad = _NW * nblk * _CHUNK
    gidx = src + (jnp.arange(27, dtype=jnp.int32) * nin)[:, None]
    gidx = jnp.concatenate([gidx.reshape(-1), jnp.zeros((n_pad - n,), jnp.int32)])
    dsti = jnp.concatenate([dst.reshape(-1), jnp.full((n_pad - n,), nout, jnp.int32)])
    return gidx.reshape(-1, _CHUNK), dsti.reshape(-1, _CHUNK), nblk


def _sc_gather_scatter_add(p_flat, gidx, dsti, co, nr_pad, nblk):
    acc_bytes = nr_pad * co * 4
    nbuf = 2
    for cand_nb in (8, 4):
        scr = 2 * (2 * cand_nb * _CHUNK * 4) + cand_nb * _CHUNK * co * 4 \
            + _RCH * co * 4
        if acc_bytes + _NS * scr <= 8_200_000:
            nbuf = cand_nb
            break
    n_ib = nblk // nbuf
    nrb = nr_pad // _RCH

    mesh = plsc.VectorSubcoreMesh(core_axis_name="c", subcore_axis_name="s")

    @functools.partial(
        pl.kernel,
        out_type=jax.ShapeDtypeStruct((_NC, nr_pad, co), jnp.float32),
        mesh=mesh,
        scratch_types=[
            pltpu.VMEM((2, nbuf, _CHUNK), jnp.int32),
            pltpu.VMEM((2, nbuf, _CHUNK), jnp.int32),
            *[pltpu.VMEM((_CHUNK, co), jnp.float32) for _ in range(nbuf)],
            pltpu.VMEM((_RCH, co), jnp.float32),
            pltpu.VMEM_SHARED((nr_pad, co), jnp.float32),
            pltpu.SemaphoreType.DMA,
            *[pltpu.SemaphoreType.DMA for _ in range(2 * nbuf)],
        ],
        compiler_params=pltpu.CompilerParams(use_tc_tiling_on_sc=False),
    )
    def k(p_hbm, gidx_hbm, dst_hbm, zero_hbm, out_hbm, idxv, dstv, *rest):
        rbufs = rest[:nbuf]
        zrow = rest[nbuf]
        acc = rest[nbuf + 1]
        isem = rest[nbuf + 2]
        gs = rest[nbuf + 3:nbuf + 3 + nbuf]
        ss = rest[nbuf + 3 + nbuf:nbuf + 3 + 2 * nbuf]
        c = lax.axis_index("c")
        s = lax.axis_index("s")
        wid = c * _NS + s

        pltpu.sync_copy(zero_hbm, zrow)

        def zloop(i, carry):
            blk = i * _NS + s

            @pl.when(blk < nrb)
            def _():
                pltpu.sync_copy(zrow, acc.at[pl.ds(blk * _RCH, _RCH)])

            return carry

        lax.fori_loop(0, -(-nrb // _NS), zloop, 0)
        plsc.subcore_barrier()

        pltpu.async_copy(gidx_hbm.at[pl.ds(wid * nblk, nbuf)], idxv.at[0], isem)
        pltpu.async_copy(dst_hbm.at[pl.ds(wid * nblk, nbuf)], dstv.at[0], isem)

        def outer(b, carry):
            p = b % 2
            pltpu.make_async_copy(
                gidx_hbm.at[pl.ds(wid * nblk, nbuf)], idxv.at[p], isem).wait()
            pltpu.make_async_copy(
                dst_hbm.at[pl.ds(wid * nblk, nbuf)], dstv.at[p], isem).wait()
            for t in range(nbuf):
                @pl.when(b > 0)
                def _():
                    pltpu.make_async_copy(
                        rbufs[t], acc.at[dstv.at[1 - p, t]], ss[t]).wait()
                pltpu.async_copy(p_hbm.at[idxv.at[p, t]], rbufs[t], gs[t])
            @pl.when(b + 1 < n_ib)
            def _():
                row0 = wid * nblk + (b + 1) * nbuf
                pltpu.async_copy(gidx_hbm.at[pl.ds(row0, nbuf)],
                                 idxv.at[1 - p], isem)
                pltpu.async_copy(dst_hbm.at[pl.ds(row0, nbuf)],
                                 dstv.at[1 - p], isem)
            for t in range(nbuf):
                pltpu.make_async_copy(
                    p_hbm.at[idxv.at[p, t]], rbufs[t], gs[t]).wait()
                pltpu.async_copy(rbufs[t], acc.at[dstv.at[p, t]], ss[t],
                                 add=True)
            return carry

        lax.fori_loop(0, n_ib, outer, 0)
        pl_last = (n_ib - 1) % 2
        for t in range(nbuf):
            pltpu.make_async_copy(
                rbufs[t], acc.at[dstv.at[pl_last, t]], ss[t]).wait()
        plsc.subcore_barrier()

        def cploop(i, carry):
            blk = i * _NS + s

            @pl.when(blk < nrb)
            def _():
                pltpu.sync_copy(acc.at[pl.ds(blk * _RCH, _RCH)],
                                out_hbm.at[c, pl.ds(blk * _RCH, _RCH)])

            return carry

        lax.fori_loop(0, -(-nrb // _NS), cploop, 0)

    zero = jnp.zeros((_RCH, co), jnp.float32)
    return k(p_flat, gidx, dsti, zero)


def _matmul_p(x, w):
    nin, ci = x.shape
    co = w.shape[2]
    bm = min(1024, nin)
    gm = -(-nin // bm)

    def mm(x_ref, w_ref, o_ref):
        o_ref[0] = jnp.dot(x_ref[...], w_ref[0],
                           preferred_element_type=jnp.float32)

    p = pl.pallas_call(
        mm,
        grid=(gm, 27),
        in_specs=[pl.BlockSpec((bm, ci), lambda i, k: (i, 0)),
                  pl.BlockSpec((1, ci, co), lambda i, k: (k, 0, 0))],
        out_specs=pl.BlockSpec((1, bm, co), lambda i, k: (k, i, 0)),
        out_shape=jax.ShapeDtypeStruct((27, nin, co), jnp.float32),
    )(x, w)
    return p.reshape(27 * nin, co)


_BM = 4096


def _bn_relu_concat(parts, g, b, nout, skip):
    cop = parts.shape[2]
    co = g.shape[0]
    cs = 0 if skip is None else skip.shape[1]
    nb = -(-nout // _BM)

    def body(p_ref, g_ref, b_ref, *rest):
        o_ref, accv = rest[-2], rest[-1]
        ph = pl.program_id(0)
        i = pl.program_id(1)
        xt = (p_ref[0] + p_ref[1])[:, :co]
        rows = i * _BM + lax.broadcasted_iota(jnp.int32, (_BM, 1), 0)
        mask = rows < nout

        @pl.when(ph == 0)
        def _():
            xm = jnp.where(mask, xt, 0.0)

            @pl.when(i == 0)
            def _():
                accv[...] = jnp.zeros_like(accv)

            accv[0, :] += jnp.sum(xm, axis=0)
            accv[1, :] += jnp.sum(xm * xm, axis=0)

        @pl.when(ph == 1)
        def _():
            m = accv[0, :] * (1.0 / nout)
            v = accv[1, :] * (1.0 / nout) - m * m
            y = (xt - m[None]) * lax.rsqrt(v[None] + 1e-5) * g_ref[...] + b_ref[...]
            y = jnp.maximum(y, 0.0)
            if cs:
                o_ref[...] = jnp.concatenate([y, rest[0][...]], axis=1)
            else:
                o_ref[...] = y

    args = [parts, g.reshape(1, co), b.reshape(1, co)]
    in_specs = [pl.BlockSpec((2, _BM, cop), lambda ph, i: (0, i, 0)),
                pl.BlockSpec((1, co), lambda ph, i: (0, 0)),
                pl.BlockSpec((1, co), lambda ph, i: (0, 0))]
    if cs:
        args.append(skip)
        in_specs.append(pl.BlockSpec((_BM, cs), lambda ph, i: (i, 0)))
    return pl.pallas_call(
        body,
        grid=(2, nb),
        in_specs=in_specs,
        out_specs=pl.BlockSpec((_BM, co + cs), lambda ph, i: (i, 0)),
        out_shape=jax.ShapeDtypeStruct((nout, co + cs), jnp.float32),
        scratch_shapes=[pltpu.VMEM((2, co), jnp.float32)],
    )(*args)


def _final_add(parts, nout, co):
    cop = parts.shape[2]
    nb = -(-nout // _BM)

    def body(p_ref, o_ref):
        o_ref[...] = (p_ref[0] + p_ref[1])[:, :co]

    return pl.pallas_call(
        body,
        grid=(nb,),
        in_specs=[pl.BlockSpec((2, _BM, cop), lambda i: (0, i, 0))],
        out_specs=pl.BlockSpec((_BM, co), lambda i: (i, 0)),
        out_shape=jax.ShapeDtypeStruct((nout, co), jnp.float32),
    )(parts)


def _layer(x, src, dst, w, nout, g=None, b=None, skip=None):
    nin = x.shape[0]
    co = w.shape[2]
    if co < 32:
        w = jnp.concatenate(
            [w, jnp.zeros((27, w.shape[1], 32 - co), jnp.float32)], axis=2)
    cop = w.shape[2]
    p = _matmul_p(x, w)
    gidx, dsti, nblk = _prep_idx(src, dst, nin, nout)
    nr_pad = -(-(nout + 1) // _RCH) * _RCH
    parts = _sc_gather_scatter_add(p, gidx, dsti, cop, nr_pad, nblk)
    if g is None:
        return _final_add(parts, nout, co)
    return _bn_relu_concat(parts, g, b, nout, skip)


def kernel(x, src0, dst0, src1, dst1, src2, dst2, src3, dst3, src4, dst4,
           srct4, dstt4, srct3, dstt3, srct2, dstt2, srct1, dstt1,
           srct0, dstt0, W0, W1, W2, W3, W4, W5, W6, W7, W8, W9,
           g0, g1, g2, g3, g4, g5, g6, g7, g8,
           b0, b1, b2, b3, b4, b5, b6, b7, b8):
    s1 = _layer(x, src0, dst0, W0, 50000, g0, b0)
    s2 = _layer(s1, src1, dst1, W1, 12500, g1, b1)
    s4 = _layer(s2, src2, dst2, W2, 3125, g2, b2)
    s8 = _layer(s4, src3, dst3, W3, 782, g3, b3)
    s16 = _layer(s8, src4, dst4, W4, 196, g4, b4)
    o = _layer(s16, srct4, dstt4, W5, 782, g5, b5, skip=s8)
    o = _layer(o, srct3, dstt3, W6, 3125, g6, b6, skip=s4)
    o = _layer(o, srct2, dstt2, W7, 12500, g7, b7, skip=s2)
    o = _layer(o, srct1, dstt1, W8, 50000, g8, b8, skip=s1)
    return _layer(o, srct0, dstt0, W9, 50000)


# speedup vs baseline: 1.0529x; 1.0529x over previous
import functools

import jax
import jax.numpy as jnp
from jax import lax
from jax.experimental import pallas as pl
from jax.experimental.pallas import tpu as pltpu
from jax.experimental.pallas import tpu_sc as plsc

_NC, _NS = 2, 16
_NW = _NC * _NS
_CHUNK = 128
_IBR = 8
_RCH = 128


def _prep_idx(src, dst, nin, nout):
    n = 27 * nout
    nblk = -(-n // (_NW * _CHUNK))
    nblk = -(-nblk // _IBR) * _IBR
    n_pad = _NW * nblk * _CHUNK
    gidx = src + (jnp.arange(27, dtype=jnp.int32) * nin)[:, None]
    gidx = jnp.concatenate([gidx.reshape(-1), jnp.zeros((n_pad - n,), jnp.int32)])
    dsti = jnp.concatenate([dst.reshape(-1), jnp.full((n_pad - n,), nout, jnp.int32)])
    return gidx.reshape(-1, _CHUNK), dsti.reshape(-1, _CHUNK), nblk


def _sc_gather_scatter_add(p_flat, gidx, dsti, co, nr_pad, nblk):
    acc_bytes = nr_pad * co * 4
    nbuf = 2
    for cand_nb in (8, 4):
        scr = 2 * (2 * cand_nb * _CHUNK * 4) + cand_nb * _CHUNK * co * 4 \
            + _RCH * co * 4
        if acc_bytes + _NS * scr <= 8_200_000:
            nbuf = cand_nb
            break
    n_ib = nblk // nbuf
    nrb = nr_pad // _RCH

    mesh = plsc.VectorSubcoreMesh(core_axis_name="c", subcore_axis_name="s")

    @functools.partial(
        pl.kernel,
        out_type=jax.ShapeDtypeStruct((_NC, nr_pad, co), jnp.float32),
        mesh=mesh,
        scratch_types=[
            pltpu.VMEM((2, nbuf, _CHUNK), jnp.int32),
            pltpu.VMEM((2, nbuf, _CHUNK), jnp.int32),
            *[pltpu.VMEM((_CHUNK, co), jnp.float32) for _ in range(nbuf)],
            pltpu.VMEM((_RCH, co), jnp.float32),
            pltpu.VMEM_SHARED((nr_pad, co), jnp.float32),
            pltpu.SemaphoreType.DMA,
            *[pltpu.SemaphoreType.DMA for _ in range(2 * nbuf)],
        ],
        compiler_params=pltpu.CompilerParams(use_tc_tiling_on_sc=False),
    )
    def k(p_hbm, gidx_hbm, dst_hbm, zero_hbm, out_hbm, idxv, dstv, *rest):
        rbufs = rest[:nbuf]
        zrow = rest[nbuf]
        acc = rest[nbuf + 1]
        isem = rest[nbuf + 2]
        gs = rest[nbuf + 3:nbuf + 3 + nbuf]
        ss = rest[nbuf + 3 + nbuf:nbuf + 3 + 2 * nbuf]
        c = lax.axis_index("c")
        s = lax.axis_index("s")
        wid = c * _NS + s

        pltpu.sync_copy(zero_hbm, zrow)

        def zloop(i, carry):
            blk = i * _NS + s

            @pl.when(blk < nrb)
            def _():
                pltpu.sync_copy(zrow, acc.at[pl.ds(blk * _RCH, _RCH)])

            return carry

        lax.fori_loop(0, -(-nrb // _NS), zloop, 0)
        plsc.subcore_barrier()

        pltpu.async_copy(gidx_hbm.at[pl.ds(wid * nblk, nbuf)], idxv.at[0], isem)
        pltpu.async_copy(dst_hbm.at[pl.ds(wid * nblk, nbuf)], dstv.at[0], isem)

        def outer(b, carry):
            p = b % 2
            pltpu.make_async_copy(
                gidx_hbm.at[pl.ds(wid * nblk, nbuf)], idxv.at[p], isem).wait()
            pltpu.make_async_copy(
                dst_hbm.at[pl.ds(wid * nblk, nbuf)], dstv.at[p], isem).wait()
            for t in range(nbuf):
                @pl.when(b > 0)
                def _():
                    pltpu.make_async_copy(
                        rbufs[t], acc.at[dstv.at[1 - p, t]], ss[t]).wait()
                pltpu.async_copy(p_hbm.at[idxv.at[p, t]], rbufs[t], gs[t])
            @pl.when(b + 1 < n_ib)
            def _():
                row0 = wid * nblk + (b + 1) * nbuf
                pltpu.async_copy(gidx_hbm.at[pl.ds(row0, nbuf)],
                                 idxv.at[1 - p], isem)
                pltpu.async_copy(dst_hbm.at[pl.ds(row0, nbuf)],
                                 dstv.at[1 - p], isem)
            for t in range(nbuf):
                pltpu.make_async_copy(
                    p_hbm.at[idxv.at[p, t]], rbufs[t], gs[t]).wait()
                pltpu.async_copy(rbufs[t], acc.at[dstv.at[p, t]], ss[t],
                                 add=True)
            return carry

        lax.fori_loop(0, n_ib, outer, 0)
        pl_last = (n_ib - 1) % 2
        for t in range(nbuf):
            pltpu.make_async_copy(
                rbufs[t], acc.at[dstv.at[pl_last, t]], ss[t]).wait()
        plsc.subcore_barrier()

        def cploop(i, carry):
            blk = i * _NS + s

            @pl.when(blk < nrb)
            def _():
                pltpu.sync_copy(acc.at[pl.ds(blk * _RCH, _RCH)],
                                out_hbm.at[c, pl.ds(blk * _RCH, _RCH)])

            return carry

        lax.fori_loop(0, -(-nrb // _NS), cploop, 0)

    zero = jnp.zeros((_RCH, co), jnp.float32)
    return k(p_flat, gidx, dsti, zero)


def _matmul_p(x, w):
    nin, ci = x.shape
    co = w.shape[2]
    bm = min(1024, nin)
    gm = -(-nin // bm)

    def mm(x_ref, w_ref, o_ref):
        o_ref[0] = jnp.dot(x_ref[...], w_ref[0],
                           preferred_element_type=jnp.float32)

    p = pl.pallas_call(
        mm,
        grid=(gm, 27),
        in_specs=[pl.BlockSpec((bm, ci), lambda i, k: (i, 0)),
                  pl.BlockSpec((1, ci, co), lambda i, k: (k, 0, 0))],
        out_specs=pl.BlockSpec((1, bm, co), lambda i, k: (k, i, 0)),
        out_shape=jax.ShapeDtypeStruct((27, nin, co), jnp.float32),
    )(x, w)
    return p.reshape(27 * nin, co)


_BM = 4096


def _bn_relu_concat(parts, g, b, nout, skip):
    cop = parts.shape[2]
    co = g.shape[0]
    cs = 0 if skip is None else skip.shape[1]
    nb = -(-nout // _BM)

    def body(p_ref, g_ref, b_ref, *rest):
        o_ref, accv = rest[-2], rest[-1]
        ph = pl.program_id(0)
        i = pl.program_id(1)
        xt = (p_ref[0] + p_ref[1])[:, :co]
        rows = i * _BM + lax.broadcasted_iota(jnp.int32, (_BM, 1), 0)
        mask = rows < nout

        @pl.when(ph == 0)
        def _():
            xm = jnp.where(mask, xt, 0.0)

            @pl.when(i == 0)
            def _():
                accv[...] = jnp.zeros_like(accv)

            accv[0, :] += jnp.sum(xm, axis=0)
            accv[1, :] += jnp.sum(xm * xm, axis=0)

        @pl.when(ph == 1)
        def _():
            m = accv[0, :] * (1.0 / nout)
            v = accv[1, :] * (1.0 / nout) - m * m
            y = (xt - m[None]) * lax.rsqrt(v[None] + 1e-5) * g_ref[...] + b_ref[...]
            y = jnp.maximum(y, 0.0)
            if cs:
                o_ref[...] = jnp.concatenate([y, rest[0][...]], axis=1)
            else:
                o_ref[...] = y

    args = [parts, g.reshape(1, co), b.reshape(1, co)]
    in_specs = [pl.BlockSpec((2, _BM, cop), lambda ph, i: (0, i, 0)),
                pl.BlockSpec((1, co), lambda ph, i: (0, 0)),
                pl.BlockSpec((1, co), lambda ph, i: (0, 0))]
    if cs:
        args.append(skip)
        in_specs.append(pl.BlockSpec((_BM, cs), lambda ph, i: (i, 0)))
    return pl.pallas_call(
        body,
        grid=(2, nb),
        in_specs=in_specs,
        out_specs=pl.BlockSpec((_BM, co + cs), lambda ph, i: (i, 0)),
        out_shape=jax.ShapeDtypeStruct((nout, co + cs), jnp.float32),
        scratch_shapes=[pltpu.VMEM((2, co), jnp.float32)],
    )(*args)


def _final_add(parts, nout, co):
    cop = parts.shape[2]
    nb = -(-nout // _BM)

    def body(p_ref, o_ref):
        o_ref[...] = (p_ref[0] + p_ref[1])[:, :co]

    return pl.pallas_call(
        body,
        grid=(nb,),
        in_specs=[pl.BlockSpec((2, _BM, cop), lambda i: (0, i, 0))],
        out_specs=pl.BlockSpec((_BM, co), lambda i: (i, 0)),
        out_shape=jax.ShapeDtypeStruct((nout, co), jnp.float32),
    )(parts)


def _bn_matmul_p(parts, g, b, skip, w, nin):
    cop = parts.shape[2]
    co = g.shape[0]
    cs = 0 if skip is None else skip.shape[1]
    ci = co + cs
    con = w.shape[2]
    bm = _BM
    gm = -(-nin // bm)

    def body(p_ref, g_ref, b_ref, *rest):
        w_ref = rest[-5]
        o_ref, y_ref = rest[-4], rest[-3]
        yv, accv = rest[-2], rest[-1]
        ph = pl.program_id(0)
        i = pl.program_id(1)
        rows = i * bm + lax.broadcasted_iota(jnp.int32, (bm, 1), 0)
        mask = rows < nin

        @pl.when(ph == 0)
        def _():
            xt = (p_ref[0] + p_ref[1])[:, :co]
            xm = jnp.where(mask, xt, 0.0)

            @pl.when(i == 0)
            def _():
                accv[...] = jnp.zeros_like(accv)

            accv[0, :] += jnp.sum(xm, axis=0)
            accv[1, :] += jnp.sum(xm * xm, axis=0)

        @pl.when(ph == 1)
        def _():
            xt = (p_ref[0] + p_ref[1])[:, :co]
            m = accv[0, :] * (1.0 / nin)
            v = accv[1, :] * (1.0 / nin) - m * m
            y = (xt - m[None]) * lax.rsqrt(v[None] + 1e-5) * g_ref[...] + b_ref[...]
            y = jnp.maximum(y, 0.0)
            if cs:
                y = jnp.concatenate([y, rest[0][...]], axis=1)
            y = jnp.where(mask, y, 0.0)
            yv[pl.ds(i * bm, bm), :] = y

        @pl.when(ph >= 2)
        def _():
            o_ref[0] = jnp.dot(yv[pl.ds(i * bm, bm), :], w_ref[0],
                               preferred_element_type=jnp.float32)

        @pl.when(ph == 28)
        def _():
            y_ref[...] = yv[pl.ds(i * bm, bm), :]

    args = [parts, g.reshape(1, co), b.reshape(1, co)]
    in_specs = [pl.BlockSpec((2, bm, cop), lambda ph, i: (0, i, 0)),
                pl.BlockSpec((1, co), lambda ph, i: (0, 0)),
                pl.BlockSpec((1, co), lambda ph, i: (0, 0))]
    if cs:
        args.append(skip)
        in_specs.append(pl.BlockSpec((bm, cs), lambda ph, i: (i, 0)))
    args.append(w)
    in_specs.append(pl.BlockSpec(
        (1, ci, con), lambda ph, i: (jnp.maximum(ph - 2, 0), 0, 0)))
    p, y = pl.pallas_call(
        body,
        grid=(29, gm),
        in_specs=in_specs,
        out_specs=[pl.BlockSpec((1, bm, con),
                                lambda ph, i: (jnp.maximum(ph - 2, 0), i, 0)),
                   pl.BlockSpec((bm, ci), lambda ph, i: (i, 0))],
        out_shape=[jax.ShapeDtypeStruct((27, nin, con), jnp.float32),
                   jax.ShapeDtypeStruct((nin, ci), jnp.float32)],
        scratch_shapes=[pltpu.VMEM((gm * bm, ci), jnp.float32),
                        pltpu.VMEM((2, co), jnp.float32)],
    )(*args)
    return p.reshape(27 * nin, con), y


def _conv_edges(p_flat, src, dst, nin, nout, cop):
    gidx, dsti, nblk = _prep_idx(src, dst, nin, nout)
    nr_pad = -(-(nout + 1) // _RCH) * _RCH
    return _sc_gather_scatter_add(p_flat, gidx, dsti, cop, nr_pad, nblk)


def kernel(x, src0, dst0, src1, dst1, src2, dst2, src3, dst3, src4, dst4,
           srct4, dstt4, srct3, dstt3, srct2, dstt2, srct1, dstt1,
           srct0, dstt0, W0, W1, W2, W3, W4, W5, W6, W7, W8, W9,
           g0, g1, g2, g3, g4, g5, g6, g7, g8,
           b0, b1, b2, b3, b4, b5, b6, b7, b8):
    w9 = jnp.concatenate([W9, jnp.zeros((27, 40, 6), jnp.float32)], axis=2)

    pa = _conv_edges(_matmul_p(x, W0), src0, dst0, 50000, 50000, 8)
    p, s1 = _bn_matmul_p(pa, g0, b0, None, W1, 50000)
    pa = _conv_edges(p, src1, dst1, 50000, 12500, 16)
    p, s2 = _bn_matmul_p(pa, g1, b1, None, W2, 12500)
    pa = _conv_edges(p, src2, dst2, 12500, 3125, 32)
    p, s4 = _bn_matmul_p(pa, g2, b2, None, W3, 3125)
    pa = _conv_edges(p, src3, dst3, 3125, 782, 64)
    p, s8 = _bn_matmul_p(pa, g3, b3, None, W4, 782)
    pa = _conv_edges(p, src4, dst4, 782, 196, 128)
    p, _ = _bn_matmul_p(pa, g4, b4, None, W5, 196)
    pa = _conv_edges(p, srct4, dstt4, 196, 782, 64)
    p, _ = _bn_matmul_p(pa, g5, b5, s8, W6, 782)
    pa = _conv_edges(p, srct3, dstt3, 782, 3125, 64)
    p, _ = _bn_matmul_p(pa, g6, b6, s4, W7, 3125)
    pa = _conv_edges(p, srct2, dstt2, 3125, 12500, 48)
    p, _ = _bn_matmul_p(pa, g7, b7, s2, W8, 12500)
    pa = _conv_edges(p, srct1, dstt1, 12500, 50000, 32)
    p, _ = _bn_matmul_p(pa, g8, b8, s1, w9, 50000)
    pa = _conv_edges(p, srct0, dstt0, 50000, 50000, 8)
    return _final_add(pa, 50000, 2)


# speedup vs baseline: 1.2317x; 1.1698x over previous
import functools

import jax
import jax.numpy as jnp
from jax import lax
from jax.experimental import pallas as pl
from jax.experimental.pallas import tpu as pltpu
from jax.experimental.pallas import tpu_sc as plsc

_NC, _NS = 2, 16
_NW = _NC * _NS
_CHUNK = 128
_IBR = 8
_RCH = 128


def _prep_idx(src, dst, nin, nout):
    n = 27 * nout
    nblk = -(-n // (_NW * _CHUNK))
    nblk = -(-nblk // _IBR) * _IBR
    n_pad = _NW * nblk * _CHUNK
    gidx = src + (jnp.arange(27, dtype=jnp.int32) * nin)[:, None]
    gidx = jnp.concatenate([gidx.reshape(-1), jnp.zeros((n_pad - n,), jnp.int32)])
    dsti = jnp.concatenate([dst.reshape(-1), jnp.full((n_pad - n,), nout, jnp.int32)])
    return gidx.reshape(-1, _CHUNK), dsti.reshape(-1, _CHUNK), nblk


def _sc_gather_scatter_add(p_flat, gidx, dsti, co, nr_pad, nblk):
    acc_bytes = nr_pad * co * 4
    nbuf = 2
    for cand_nb in (8, 4):
        scr = 2 * (2 * cand_nb * _CHUNK * 4) + cand_nb * _CHUNK * co * 4 \
            + _RCH * co * 4
        if acc_bytes + _NS * scr <= 8_200_000:
            nbuf = cand_nb
            break
    n_ib = nblk // nbuf
    nrb = nr_pad // _RCH

    mesh = plsc.VectorSubcoreMesh(core_axis_name="c", subcore_axis_name="s")

    @functools.partial(
        pl.kernel,
        out_type=jax.ShapeDtypeStruct((_NC, nr_pad, co), jnp.float32),
        mesh=mesh,
        scratch_types=[
            pltpu.VMEM((2, nbuf, _CHUNK), jnp.int32),
            pltpu.VMEM((2, nbuf, _CHUNK), jnp.int32),
            *[pltpu.VMEM((_CHUNK, co), jnp.float32) for _ in range(nbuf)],
            pltpu.VMEM((_RCH, co), jnp.float32),
            pltpu.VMEM_SHARED((nr_pad, co), jnp.float32),
            pltpu.SemaphoreType.DMA,
            *[pltpu.SemaphoreType.DMA for _ in range(2 * nbuf)],
        ],
        compiler_params=pltpu.CompilerParams(use_tc_tiling_on_sc=False),
    )
    def k(p_hbm, gidx_hbm, dst_hbm, zero_hbm, out_hbm, idxv, dstv, *rest):
        rbufs = rest[:nbuf]
        zrow = rest[nbuf]
        acc = rest[nbuf + 1]
        isem = rest[nbuf + 2]
        gs = rest[nbuf + 3:nbuf + 3 + nbuf]
        ss = rest[nbuf + 3 + nbuf:nbuf + 3 + 2 * nbuf]
        c = lax.axis_index("c")
        s = lax.axis_index("s")
        wid = c * _NS + s

        pltpu.sync_copy(zero_hbm, zrow)

        def zloop(i, carry):
            blk = i * _NS + s

            @pl.when(blk < nrb)
            def _():
                pltpu.sync_copy(zrow, acc.at[pl.ds(blk * _RCH, _RCH)])

            return carry

        lax.fori_loop(0, -(-nrb // _NS), zloop, 0)
        plsc.subcore_barrier()

        pltpu.async_copy(gidx_hbm.at[pl.ds(wid * nblk, nbuf)], idxv.at[0], isem)
        pltpu.async_copy(dst_hbm.at[pl.ds(wid * nblk, nbuf)], dstv.at[0], isem)

        def outer(b, carry):
            p = b % 2
            pltpu.make_async_copy(
                gidx_hbm.at[pl.ds(wid * nblk, nbuf)], idxv.at[p], isem).wait()
            pltpu.make_async_copy(
                dst_hbm.at[pl.ds(wid * nblk, nbuf)], dstv.at[p], isem).wait()
            for t in range(nbuf):
                @pl.when(b > 0)
                def _():
                    pltpu.make_async_copy(
                        rbufs[t], acc.at[dstv.at[1 - p, t]], ss[t]).wait()
                pltpu.async_copy(p_hbm.at[idxv.at[p, t]], rbufs[t], gs[t])
            @pl.when(b + 1 < n_ib)
            def _():
                row0 = wid * nblk + (b + 1) * nbuf
                pltpu.async_copy(gidx_hbm.at[pl.ds(row0, nbuf)],
                                 idxv.at[1 - p], isem)
                pltpu.async_copy(dst_hbm.at[pl.ds(row0, nbuf)],
                                 dstv.at[1 - p], isem)
            for t in range(nbuf):
                pltpu.make_async_copy(
                    p_hbm.at[idxv.at[p, t]], rbufs[t], gs[t]).wait()
                pltpu.async_copy(rbufs[t], acc.at[dstv.at[p, t]], ss[t],
                                 add=True)
            return carry

        lax.fori_loop(0, n_ib, outer, 0)
        pl_last = (n_ib - 1) % 2
        for t in range(nbuf):
            pltpu.make_async_copy(
                rbufs[t], acc.at[dstv.at[pl_last, t]], ss[t]).wait()
        plsc.subcore_barrier()

        def cploop(i, carry):
            blk = i * _NS + s

            @pl.when(blk < nrb)
            def _():
                pltpu.sync_copy(acc.at[pl.ds(blk * _RCH, _RCH)],
                                out_hbm.at[c, pl.ds(blk * _RCH, _RCH)])

            return carry

        lax.fori_loop(0, -(-nrb // _NS), cploop, 0)

    zero = jnp.zeros((_RCH, co), jnp.float32)
    return k(p_flat, gidx, dsti, zero)


def _matmul_p(x, w):
    nin, ci = x.shape
    co = w.shape[2]
    bm = min(1024, nin)
    gm = -(-nin // bm)

    def mm(x_ref, w_ref, o_ref):
        o_ref[0] = jnp.dot(x_ref[...], w_ref[0],
                           preferred_element_type=jnp.float32)

    p = pl.pallas_call(
        mm,
        grid=(gm, 27),
        in_specs=[pl.BlockSpec((bm, ci), lambda i, k: (i, 0)),
                  pl.BlockSpec((1, ci, co), lambda i, k: (k, 0, 0))],
        out_specs=pl.BlockSpec((1, bm, co), lambda i, k: (k, i, 0)),
        out_shape=jax.ShapeDtypeStruct((27, nin, co), jnp.float32),
    )(x, w)
    return p.reshape(27 * nin, co)


_BM = 4096


def _bn_relu_concat(parts, g, b, nout, skip):
    cop = parts.shape[2]
    co = g.shape[0]
    cs = 0 if skip is None else skip.shape[1]
    nb = -(-nout // _BM)

    def body(p_ref, g_ref, b_ref, *rest):
        o_ref, accv = rest[-2], rest[-1]
        ph = pl.program_id(0)
        i = pl.program_id(1)
        xt = (p_ref[0] + p_ref[1])[:, :co]
        rows = i * _BM + lax.broadcasted_iota(jnp.int32, (_BM, 1), 0)
        mask = rows < nout

        @pl.when(ph == 0)
        def _():
            xm = jnp.where(mask, xt, 0.0)

            @pl.when(i == 0)
            def _():
                accv[...] = jnp.zeros_like(accv)

            accv[0, :] += jnp.sum(xm, axis=0)
            accv[1, :] += jnp.sum(xm * xm, axis=0)

        @pl.when(ph == 1)
        def _():
            m = accv[0, :] * (1.0 / nout)
            v = accv[1, :] * (1.0 / nout) - m * m
            y = (xt - m[None]) * lax.rsqrt(v[None] + 1e-5) * g_ref[...] + b_ref[...]
            y = jnp.maximum(y, 0.0)
            if cs:
                o_ref[...] = jnp.concatenate([y, rest[0][...]], axis=1)
            else:
                o_ref[...] = y

    args = [parts, g.reshape(1, co), b.reshape(1, co)]
    in_specs = [pl.BlockSpec((2, _BM, cop), lambda ph, i: (0, i, 0)),
                pl.BlockSpec((1, co), lambda ph, i: (0, 0)),
                pl.BlockSpec((1, co), lambda ph, i: (0, 0))]
    if cs:
        args.append(skip)
        in_specs.append(pl.BlockSpec((_BM, cs), lambda ph, i: (i, 0)))
    return pl.pallas_call(
        body,
        grid=(2, nb),
        in_specs=in_specs,
        out_specs=pl.BlockSpec((_BM, co + cs), lambda ph, i: (i, 0)),
        out_shape=jax.ShapeDtypeStruct((nout, co + cs), jnp.float32),
        scratch_shapes=[pltpu.VMEM((2, co), jnp.float32)],
    )(*args)


def _final_add(parts, nout, co):
    cop = parts.shape[2]
    nb = -(-nout // _BM)

    def body(p_ref, o_ref):
        o_ref[...] = (p_ref[0] + p_ref[1])[:, :co]

    return pl.pallas_call(
        body,
        grid=(nb,),
        in_specs=[pl.BlockSpec((2, _BM, cop), lambda i: (0, i, 0))],
        out_specs=pl.BlockSpec((_BM, co), lambda i: (i, 0)),
        out_shape=jax.ShapeDtypeStruct((nout, co), jnp.float32),
    )(parts)


def _bn_matmul_p(parts, g, b, skip, w, nin):
    cop = parts.shape[2]
    co = g.shape[0]
    cs = 0 if skip is None else skip.shape[1]
    ci = co + cs
    con = w.shape[2]
    bm = _BM
    gm = -(-nin // bm)

    def body(p_ref, g_ref, b_ref, *rest):
        w_ref = rest[-5]
        o_ref, y_ref = rest[-4], rest[-3]
        yv, accv = rest[-2], rest[-1]
        ph = pl.program_id(0)
        i = pl.program_id(1)
        rows = i * bm + lax.broadcasted_iota(jnp.int32, (bm, 1), 0)
        mask = rows < nin

        @pl.when(ph == 0)
        def _():
            xt = (p_ref[0] + p_ref[1])[:, :co]
            xm = jnp.where(mask, xt, 0.0)

            @pl.when(i == 0)
            def _():
                accv[...] = jnp.zeros_like(accv)

            accv[0, :] += jnp.sum(xm, axis=0)
            accv[1, :] += jnp.sum(xm * xm, axis=0)

        @pl.when(ph == 1)
        def _():
            xt = (p_ref[0] + p_ref[1])[:, :co]
            m = accv[0, :] * (1.0 / nin)
            v = accv[1, :] * (1.0 / nin) - m * m
            y = (xt - m[None]) * lax.rsqrt(v[None] + 1e-5) * g_ref[...] + b_ref[...]
            y = jnp.maximum(y, 0.0)
            if cs:
                y = jnp.concatenate([y, rest[0][...]], axis=1)
            y = jnp.where(mask, y, 0.0)
            yv[pl.ds(i * bm, bm), :] = y

        @pl.when(ph >= 2)
        def _():
            o_ref[0] = jnp.dot(yv[pl.ds(i * bm, bm), :], w_ref[0],
                               preferred_element_type=jnp.float32)

        @pl.when(ph == 28)
        def _():
            y_ref[...] = yv[pl.ds(i * bm, bm), :]

    args = [parts, g.reshape(1, co), b.reshape(1, co)]
    in_specs = [pl.BlockSpec((2, bm, cop),
                             lambda ph, i: (0, jnp.where(ph < 2, i, 0), 0)),
                pl.BlockSpec((1, co), lambda ph, i: (0, 0)),
                pl.BlockSpec((1, co), lambda ph, i: (0, 0))]
    if cs:
        args.append(skip)
        in_specs.append(pl.BlockSpec(
            (bm, cs), lambda ph, i: (jnp.where(ph < 2, i, 0), 0)))
    args.append(w)
    in_specs.append(pl.BlockSpec(
        (1, ci, con), lambda ph, i: (jnp.maximum(ph - 2, 0), 0, 0)))
    p, y = pl.pallas_call(
        body,
        grid=(29, gm),
        in_specs=in_specs,
        out_specs=[pl.BlockSpec((1, bm, con),
                                lambda ph, i: (jnp.maximum(ph - 2, 0), i, 0)),
                   pl.BlockSpec((bm, ci), lambda ph, i: (i, 0))],
        out_shape=[jax.ShapeDtypeStruct((27, nin, con), jnp.float32),
                   jax.ShapeDtypeStruct((nin, ci), jnp.float32)],
        scratch_shapes=[pltpu.VMEM((gm * bm, ci), jnp.float32),
                        pltpu.VMEM((2, co), jnp.float32)],
    )(*args)
    return p.reshape(27 * nin, con), y


def _conv_edges(p_flat, src, dst, nin, nout, cop):
    gidx, dsti, nblk = _prep_idx(src, dst, nin, nout)
    nr_pad = -(-(nout + 1) // _RCH) * _RCH
    return _sc_gather_scatter_add(p_flat, gidx, dsti, cop, nr_pad, nblk)


def kernel(x, src0, dst0, src1, dst1, src2, dst2, src3, dst3, src4, dst4,
           srct4, dstt4, srct3, dstt3, srct2, dstt2, srct1, dstt1,
           srct0, dstt0, W0, W1, W2, W3, W4, W5, W6, W7, W8, W9,
           g0, g1, g2, g3, g4, g5, g6, g7, g8,
           b0, b1, b2, b3, b4, b5, b6, b7, b8):
    w9 = jnp.concatenate([W9, jnp.zeros((27, 40, 6), jnp.float32)], axis=2)

    pa = _conv_edges(_matmul_p(x, W0), src0, dst0, 50000, 50000, 8)
    p, s1 = _bn_matmul_p(pa, g0, b0, None, W1, 50000)
    pa = _conv_edges(p, src1, dst1, 50000, 12500, 16)
    p, s2 = _bn_matmul_p(pa, g1, b1, None, W2, 12500)
    pa = _conv_edges(p, src2, dst2, 12500, 3125, 32)
    p, s4 = _bn_matmul_p(pa, g2, b2, None, W3, 3125)
    pa = _conv_edges(p, src3, dst3, 3125, 782, 64)
    p, s8 = _bn_matmul_p(pa, g3, b3, None, W4, 782)
    pa = _conv_edges(p, src4, dst4, 782, 196, 128)
    p, _ = _bn_matmul_p(pa, g4, b4, None, W5, 196)
    pa = _conv_edges(p, srct4, dstt4, 196, 782, 64)
    p, _ = _bn_matmul_p(pa, g5, b5, s8, W6, 782)
    pa = _conv_edges(p, srct3, dstt3, 782, 3125, 64)
    p, _ = _bn_matmul_p(pa, g6, b6, s4, W7, 3125)
    pa = _conv_edges(p, srct2, dstt2, 3125, 12500, 48)
    p, _ = _bn_matmul_p(pa, g7, b7, s2, W8, 12500)
    pa = _conv_edges(p, srct1, dstt1, 12500, 50000, 32)
    p, _ = _bn_matmul_p(pa, g8, b8, s1, w9, 50000)
    pa = _conv_edges(p, srct0, dstt0, 50000, 50000, 8)
    return _final_add(pa, 50000, 2)
